# baseline (speedup 1.0000x reference)
import jax
import jax.numpy as jnp
from jax import lax
from jax.experimental import pallas as pl
from jax.experimental.pallas import tpu as pltpu

N_FEAT = 10000
N_OUT = 1024
N_P = 16
N_ROWS = 4096
OUT_B = 8
OUT_H = 128

BM = 512
BK = 5000
KBLOCKS = N_FEAT // BK
MBLOCKS = N_ROWS // BM


def _tc_body(idx_ref, w_ref, xt_ref, out_ref, m_scr):
    m = pl.program_id(0)
    k = pl.program_id(1)
    base = k * BK

    def _build_slab(slab_base):
        riota = (lax.broadcasted_iota(jnp.int16, (BK, N_OUT), 0)
                 + jnp.int16(slab_base))
        acc = jnp.zeros((BK, N_OUT), jnp.bfloat16)
        zero = jnp.zeros((BK, N_OUT), jnp.bfloat16)
        for p in range(N_P):
            ip = idx_ref[pl.ds(p, 1), :].astype(jnp.int16)
            wp = w_ref[pl.ds(p, 1), :].astype(jnp.bfloat16)
            acc = acc + jnp.where(riota == ip, wp, zero)
        m_scr[pl.ds(slab_base, BK), :] = acc

    @pl.when((m == 0) & (k == 0))
    def _build_first():
        _build_slab(0)

    xb = xt_ref[...].astype(jnp.bfloat16)
    prod = lax.dot_general(xb, m_scr[pl.ds(base, BK), :],
                           (((0,), (0,)), ((), ())),
                           preferred_element_type=jnp.float32)

    @pl.when(k == 0)
    def _init():
        out_ref[...] = prod

    @pl.when(k != 0)
    def _accum():
        out_ref[...] = out_ref[...] + prod

    @pl.when((m == 0) & (k == 0))
    def _build_second():
        _build_slab(BK)


_branch_tc = pl.pallas_call(
    _tc_body,
    grid=(MBLOCKS, KBLOCKS),
    in_specs=[
        pl.BlockSpec((N_P, N_OUT), lambda m, k: (0, 0)),
        pl.BlockSpec((N_P, N_OUT), lambda m, k: (0, 0)),
        pl.BlockSpec((BK, BM), lambda m, k: (k, m)),
    ],
    out_specs=pl.BlockSpec((BM, N_OUT), lambda m, k: (m, 0)),
    out_shape=jax.ShapeDtypeStruct((N_ROWS, N_OUT), jnp.float32),
    scratch_shapes=[pltpu.VMEM((N_FEAT, N_OUT), jnp.bfloat16)],
    compiler_params=pltpu.CompilerParams(
        dimension_semantics=("arbitrary", "arbitrary")),
)


def kernel(x, weights, all_branch_indices):
    out = _branch_tc(all_branch_indices, weights, x.T)
    return out.reshape(N_ROWS, OUT_B, OUT_H)


# speedup vs baseline: 1.2887x; 1.2887x over previous
import jax
import jax.numpy as jnp
from jax import lax
from jax.experimental import pallas as pl
from jax.experimental.pallas import tpu as pltpu

N_FEAT = 10000
N_OUT = 1024
N_P = 16
N_ROWS = 4096
OUT_B = 8
OUT_H = 128

BM = 512
BK = 5000
KBLOCKS = N_FEAT // BK
MBLOCKS = N_ROWS // BM


def _tc_body(idx_ref, w_ref, xt_ref, out_ref, m_scr):
    m = pl.program_id(0)
    k = pl.program_id(1)
    base = k * BK

    @pl.when(m == 0)
    def _build():
        base16 = lax.convert_element_type(base, jnp.int16)
        riota = lax.broadcasted_iota(jnp.int16, (BK, N_OUT), 0) + base16
        acc = jnp.zeros((BK, N_OUT), jnp.bfloat16)
        zero = jnp.zeros((BK, N_OUT), jnp.bfloat16)
        for p in range(N_P):
            ip = idx_ref[pl.ds(p, 1), :].astype(jnp.int16)
            wp = w_ref[pl.ds(p, 1), :].astype(jnp.bfloat16)
            acc = acc + jnp.where(riota == ip, wp, zero)
        m_scr[pl.ds(base, BK), :] = acc

    xb = xt_ref[...].astype(jnp.bfloat16)
    prod = lax.dot_general(xb, m_scr[pl.ds(base, BK), :],
                           (((0,), (0,)), ((), ())),
                           preferred_element_type=jnp.float32)

    @pl.when(k == 0)
    def _init():
        out_ref[...] = prod

    @pl.when(k != 0)
    def _accum():
        out_ref[...] = out_ref[...] + prod


_branch_tc = pl.pallas_call(
    _tc_body,
    grid=(MBLOCKS, KBLOCKS),
    in_specs=[
        pl.BlockSpec((N_P, N_OUT), lambda m, k: (0, 0)),
        pl.BlockSpec((N_P, N_OUT), lambda m, k: (0, 0)),
        pl.BlockSpec((BK, BM), lambda m, k: (k, m)),
    ],
    out_specs=pl.BlockSpec((BM, N_OUT), lambda m, k: (m, 0)),
    out_shape=jax.ShapeDtypeStruct((N_ROWS, N_OUT), jnp.float32),
    scratch_shapes=[pltpu.VMEM((N_FEAT, N_OUT), jnp.bfloat16)],
    compiler_params=pltpu.CompilerParams(
        dimension_semantics=("arbitrary", "arbitrary")),
)


def kernel(x, weights, all_branch_indices):
    out = _branch_tc(all_branch_indices, weights, x.T)
    return out.reshape(N_ROWS, OUT_B, OUT_H)
